# Initial kernel scaffold; baseline (speedup 1.0000x reference)
import functools

import jax
import jax.numpy as jnp
from jax import lax
from jax.experimental import pallas as pl
from jax.experimental.pallas import tpu as pltpu
from jax.experimental.pallas import tpu_sc as plsc

N_NODES = 10000
E_PER_REL = 160000
DIM = 128

NC = 2
NS = 16
CHUNK = 80
EDGES_PER_TILE = E_PER_REL // NS
N_CHUNKS = EDGES_PER_TILE // CHUNK
ROWS_PER_TILE = N_NODES // NS
DEG_W = 16


def _sc_layer_body(with_deg, *refs):
    if with_deg:
        (h_hbm, src_hbm, dst_hbm, zacc_hbm, zdeg_hbm, ones_hbm,
         acc0_hbm, acc1_hbm, deg0_hbm, deg1_hbm,
         acc_s, deg_s, idx_src, idx_dst, rowbuf, onesbuf, sem) = refs
    else:
        (h_hbm, src_hbm, dst_hbm, zacc_hbm,
         acc0_hbm, acc1_hbm,
         acc_s, idx_src, idx_dst, rowbuf, sem) = refs
        deg_s = zdeg_hbm = ones_hbm = onesbuf = None

    c = lax.axis_index("c")
    s = lax.axis_index("s")

    rslc = pl.ds(s * ROWS_PER_TILE, ROWS_PER_TILE)
    pltpu.sync_copy(zacc_hbm.at[rslc], acc_s.at[rslc])
    if with_deg:
        pltpu.sync_copy(zdeg_hbm.at[rslc], deg_s.at[rslc])
        pltpu.sync_copy(ones_hbm, onesbuf)
    plsc.subcore_barrier()

    base = (c * NS + s) * EDGES_PER_TILE

    def step(j, carry):
        off = base + j * CHUNK
        pltpu.sync_copy(src_hbm.at[pl.ds(off, CHUNK)], idx_src)
        pltpu.sync_copy(dst_hbm.at[pl.ds(off, CHUNK)], idx_dst)
        pltpu.async_copy(h_hbm.at[idx_src], rowbuf, sem).wait()
        pltpu.sync_copy(rowbuf, acc_s.at[idx_dst], add=True)
        if with_deg:
            pltpu.sync_copy(onesbuf, deg_s.at[idx_dst], add=True)
        return carry

    lax.fori_loop(0, N_CHUNKS, step, 0)
    plsc.subcore_barrier()

    @pl.when(c == 0)
    def _():
        pltpu.sync_copy(acc_s.at[rslc], acc0_hbm.at[rslc])
        if with_deg:
            pltpu.sync_copy(deg_s.at[rslc], deg0_hbm.at[rslc])

    @pl.when(c == 1)
    def _():
        pltpu.sync_copy(acc_s.at[rslc], acc1_hbm.at[rslc])
        if with_deg:
            pltpu.sync_copy(deg_s.at[rslc], deg1_hbm.at[rslc])


def _make_sc_layer(with_deg):
    mesh = plsc.VectorSubcoreMesh(core_axis_name="c", subcore_axis_name="s")
    f32 = jnp.float32
    out_type = [jax.ShapeDtypeStruct((N_NODES, DIM), f32),
                jax.ShapeDtypeStruct((N_NODES, DIM), f32)]
    scratch = [pltpu.VMEM_SHARED((N_NODES, DIM), f32)]
    if with_deg:
        out_type += [jax.ShapeDtypeStruct((N_NODES, DEG_W), f32),
                     jax.ShapeDtypeStruct((N_NODES, DEG_W), f32)]
        scratch += [pltpu.VMEM_SHARED((N_NODES, DEG_W), f32)]
    scratch += [pltpu.VMEM((CHUNK,), jnp.int32),
                pltpu.VMEM((CHUNK,), jnp.int32),
                pltpu.VMEM((CHUNK, DIM), f32)]
    if with_deg:
        scratch += [pltpu.VMEM((CHUNK, DEG_W), f32)]
    scratch += [pltpu.SemaphoreType.DMA]
    return pl.kernel(functools.partial(_sc_layer_body, with_deg),
                     out_type=out_type, mesh=mesh, scratch_types=scratch,
                     name="sc_gnn_layer_deg" if with_deg else "sc_gnn_layer")


_sc_layer_with_deg = _make_sc_layer(True)
_sc_layer = _make_sc_layer(False)


def _tc_body(relu, acc0, acc1, deg0, deg1, w0, b0, w1, b1, out):
    inv0 = 1.0 / jnp.maximum(deg0[...][:, 0:1], 1.0)
    inv1 = 1.0 / jnp.maximum(deg1[...][:, 0:1], 1.0)
    y = (jnp.dot(acc0[...] * inv0, w0[...], preferred_element_type=jnp.float32)
         + jnp.dot(acc1[...] * inv1, w1[...], preferred_element_type=jnp.float32)
         + b0[...] + b1[...])
    if relu:
        y = jnp.maximum(y, 0.0)
    out[...] = y


_TC_ROWS = 1000


def _tc_layer(relu, acc0, acc1, deg0, deg1, w0, b0, w1, b1):
    grid = (N_NODES // _TC_ROWS,)
    rb = lambda i: (i, 0)
    fix = lambda i: (0, 0)
    return pl.pallas_call(
        functools.partial(_tc_body, relu),
        grid=grid,
        in_specs=[
            pl.BlockSpec((_TC_ROWS, DIM), rb),
            pl.BlockSpec((_TC_ROWS, DIM), rb),
            pl.BlockSpec((_TC_ROWS, DEG_W), rb),
            pl.BlockSpec((_TC_ROWS, DEG_W), rb),
            pl.BlockSpec((DIM, DIM), fix),
            pl.BlockSpec((1, DIM), fix),
            pl.BlockSpec((DIM, DIM), fix),
            pl.BlockSpec((1, DIM), fix),
        ],
        out_specs=pl.BlockSpec((_TC_ROWS, DIM), rb),
        out_shape=jax.ShapeDtypeStruct((N_NODES, DIM), jnp.float32),
    )(acc0, acc1, deg0, deg1, w0, b0, w1, b1)


def kernel(x, edge_index_rel0, edge_index_rel1, W0_0, b0_0, W0_1, b0_1,
           W1_0, b1_0, W1_1, b1_1, W2_0, b2_0, W2_1, b2_1):
    f32 = jnp.float32
    src = jnp.concatenate([edge_index_rel0[0], edge_index_rel1[0]])
    dst = jnp.concatenate([edge_index_rel0[1], edge_index_rel1[1]])
    zacc = jnp.zeros((N_NODES, DIM), f32)
    zdeg = jnp.zeros((N_NODES, DEG_W), f32)
    ones = jnp.ones((CHUNK, DEG_W), f32)

    acc0, acc1, deg0, deg1 = _sc_layer_with_deg(x, src, dst, zacc, zdeg, ones)
    h = _tc_layer(True, acc0, acc1, deg0, deg1,
                  W0_0, b0_0.reshape(1, DIM), W0_1, b0_1.reshape(1, DIM))

    acc0, acc1 = _sc_layer(h, src, dst, zacc)
    h = _tc_layer(True, acc0, acc1, deg0, deg1,
                  W1_0, b1_0.reshape(1, DIM), W1_1, b1_1.reshape(1, DIM))

    acc0, acc1 = _sc_layer(h, src, dst, zacc)
    h = _tc_layer(False, acc0, acc1, deg0, deg1,
                  W2_0, b2_0.reshape(1, DIM), W2_1, b2_1.reshape(1, DIM))
    return h


# speedup vs baseline: 4.1582x; 4.1582x over previous
import functools

import jax
import jax.numpy as jnp
from jax import lax
from jax.experimental import pallas as pl
from jax.experimental.pallas import tpu as pltpu
from jax.experimental.pallas import tpu_sc as plsc

N_NODES = 10000
N_PAD = 10240
E_PER_REL = 160000
DIM = 128

NC = 2
NS = 16
CHUNK = 80
EDGES_PER_TILE = E_PER_REL // NS
N_CHUNKS = EDGES_PER_TILE // CHUNK
ROWS_PER_TILE = N_PAD // NS
DEG_W = 16


def _sc_layer_body(with_deg, *refs):
    if with_deg:
        (h_hbm, src_hbm, dst_hbm, zacc_hbm, zdeg_hbm, ones_hbm,
         acc0_hbm, acc1_hbm, deg0_hbm, deg1_hbm,
         acc_s, deg_s, idx_src, idx_dst, rowbuf, onesbuf, sem) = refs
    else:
        (h_hbm, src_hbm, dst_hbm, zacc_hbm,
         acc0_hbm, acc1_hbm,
         acc_s, idx_src, idx_dst, rowbuf, sem) = refs
        deg_s = zdeg_hbm = ones_hbm = onesbuf = None

    c = lax.axis_index("c")
    s = lax.axis_index("s")

    rslc = pl.ds(s * ROWS_PER_TILE, ROWS_PER_TILE)
    pltpu.sync_copy(zacc_hbm.at[rslc], acc_s.at[rslc])
    if with_deg:
        pltpu.sync_copy(zdeg_hbm.at[rslc], deg_s.at[rslc])
        pltpu.sync_copy(ones_hbm, onesbuf)
    plsc.subcore_barrier()

    base = (c * NS + s) * EDGES_PER_TILE

    def step(j, carry):
        off = base + j * CHUNK
        pltpu.sync_copy(src_hbm.at[pl.ds(off, CHUNK)], idx_src)
        pltpu.sync_copy(dst_hbm.at[pl.ds(off, CHUNK)], idx_dst)
        pltpu.async_copy(h_hbm.at[idx_src], rowbuf, sem).wait()
        pltpu.sync_copy(rowbuf, acc_s.at[idx_dst], add=True)
        if with_deg:
            pltpu.sync_copy(onesbuf, deg_s.at[idx_dst], add=True)
        return carry

    lax.fori_loop(0, N_CHUNKS, step, 0)
    plsc.subcore_barrier()

    @pl.when(c == 0)
    def _():
        pltpu.sync_copy(acc_s.at[rslc], acc0_hbm.at[rslc])
        if with_deg:
            pltpu.sync_copy(deg_s.at[rslc], deg0_hbm.at[rslc])

    @pl.when(c == 1)
    def _():
        pltpu.sync_copy(acc_s.at[rslc], acc1_hbm.at[rslc])
        if with_deg:
            pltpu.sync_copy(deg_s.at[rslc], deg1_hbm.at[rslc])


def _make_sc_layer(with_deg):
    mesh = plsc.VectorSubcoreMesh(core_axis_name="c", subcore_axis_name="s")
    f32 = jnp.float32
    out_type = [jax.ShapeDtypeStruct((N_PAD, DIM), f32),
                jax.ShapeDtypeStruct((N_PAD, DIM), f32)]
    scratch = [pltpu.VMEM_SHARED((N_PAD, DIM), f32)]
    if with_deg:
        out_type += [jax.ShapeDtypeStruct((N_PAD, DEG_W), f32),
                     jax.ShapeDtypeStruct((N_PAD, DEG_W), f32)]
        scratch += [pltpu.VMEM_SHARED((N_PAD, DEG_W), f32)]
    scratch += [pltpu.VMEM((CHUNK,), jnp.int32),
                pltpu.VMEM((CHUNK,), jnp.int32),
                pltpu.VMEM((CHUNK, DIM), f32)]
    if with_deg:
        scratch += [pltpu.VMEM((CHUNK, DEG_W), f32)]
    scratch += [pltpu.SemaphoreType.DMA]
    return pl.kernel(functools.partial(_sc_layer_body, with_deg),
                     out_type=out_type, mesh=mesh, scratch_types=scratch,
                     compiler_params=pltpu.CompilerParams(
                         use_tc_tiling_on_sc=False),
                     name="sc_gnn_layer_deg" if with_deg else "sc_gnn_layer")


_sc_layer_with_deg = _make_sc_layer(True)
_sc_layer = _make_sc_layer(False)


def _tc_body(relu, acc0, acc1, deg0, deg1, w0, b0, w1, b1, out):
    inv0 = 1.0 / jnp.maximum(deg0[...][:, 0:1], 1.0)
    inv1 = 1.0 / jnp.maximum(deg1[...][:, 0:1], 1.0)
    y = (jnp.dot(acc0[...] * inv0, w0[...], preferred_element_type=jnp.float32)
         + jnp.dot(acc1[...] * inv1, w1[...], preferred_element_type=jnp.float32)
         + b0[...] + b1[...])
    if relu:
        y = jnp.maximum(y, 0.0)
    out[...] = y


_TC_ROWS = 1024


def _tc_layer(relu, acc0, acc1, deg0, deg1, w0, b0, w1, b1):
    grid = (N_PAD // _TC_ROWS,)
    rb = lambda i: (i, 0)
    fix = lambda i: (0, 0)
    return pl.pallas_call(
        functools.partial(_tc_body, relu),
        grid=grid,
        in_specs=[
            pl.BlockSpec((_TC_ROWS, DIM), rb),
            pl.BlockSpec((_TC_ROWS, DIM), rb),
            pl.BlockSpec((_TC_ROWS, DEG_W), rb),
            pl.BlockSpec((_TC_ROWS, DEG_W), rb),
            pl.BlockSpec((DIM, DIM), fix),
            pl.BlockSpec((1, DIM), fix),
            pl.BlockSpec((DIM, DIM), fix),
            pl.BlockSpec((1, DIM), fix),
        ],
        out_specs=pl.BlockSpec((_TC_ROWS, DIM), rb),
        out_shape=jax.ShapeDtypeStruct((N_PAD, DIM), jnp.float32),
    )(acc0, acc1, deg0, deg1, w0, b0, w1, b1)


def kernel(x, edge_index_rel0, edge_index_rel1, W0_0, b0_0, W0_1, b0_1,
           W1_0, b1_0, W1_1, b1_1, W2_0, b2_0, W2_1, b2_1):
    f32 = jnp.float32
    src = jnp.concatenate([edge_index_rel0[0], edge_index_rel1[0]])
    dst = jnp.concatenate([edge_index_rel0[1], edge_index_rel1[1]])
    zacc = jnp.zeros((N_PAD, DIM), f32)
    zdeg = jnp.zeros((N_PAD, DEG_W), f32)
    ones = jnp.ones((CHUNK, DEG_W), f32)

    acc0, acc1, deg0, deg1 = _sc_layer_with_deg(x, src, dst, zacc, zdeg, ones)
    h = _tc_layer(True, acc0, acc1, deg0, deg1,
                  W0_0, b0_0.reshape(1, DIM), W0_1, b0_1.reshape(1, DIM))

    acc0, acc1 = _sc_layer(h, src, dst, zacc)
    h = _tc_layer(True, acc0, acc1, deg0, deg1,
                  W1_0, b1_0.reshape(1, DIM), W1_1, b1_1.reshape(1, DIM))

    acc0, acc1 = _sc_layer(h, src, dst, zacc)
    h = _tc_layer(False, acc0, acc1, deg0, deg1,
                  W2_0, b2_0.reshape(1, DIM), W2_1, b2_1.reshape(1, DIM))
    return h[:N_NODES]


# speedup vs baseline: 10.3133x; 2.4802x over previous
import functools

import jax
import jax.numpy as jnp
from jax import lax
from jax.experimental import pallas as pl
from jax.experimental.pallas import tpu as pltpu
from jax.experimental.pallas import tpu_sc as plsc

N_NODES = 10000
N_PAD = 10240
E_PER_REL = 160000
DIM = 128

NC = 2
NS = 16
EDGES_PER_TILE = E_PER_REL // NS
ROWS_PER_TILE = N_PAD // NS
DEG_W = 16

CHUNK = 64
N_FULL = EDGES_PER_TILE // CHUNK
TAIL = EDGES_PER_TILE - N_FULL * CHUNK
NBUF = 4


def _sc_layer_body(with_deg, *refs):
    if with_deg:
        (h_hbm, src_hbm, dst_hbm, zacc_hbm, zdeg_hbm, ones_hbm,
         acc0_hbm, acc1_hbm, deg0_hbm, deg1_hbm,
         acc_s, deg_s, idxs_v, idxd_v, rowb, onesb,
         idxs_r, idxd_r, rowr, *sems) = refs
    else:
        (h_hbm, src_hbm, dst_hbm, zacc_hbm,
         acc0_hbm, acc1_hbm,
         acc_s, idxs_v, idxd_v, rowb,
         idxs_r, idxd_r, rowr, *sems) = refs
        deg_s = zdeg_hbm = ones_hbm = onesb = None
    gsem = sems[:NBUF]
    ssem = sems[NBUF:2 * NBUF]
    isem = sems[2 * NBUF:3 * NBUF]
    dsem = sems[3 * NBUF:]

    c = lax.axis_index("c")
    s = lax.axis_index("s")
    ebase = (c * NS + s) * EDGES_PER_TILE

    rslc = pl.ds(s * ROWS_PER_TILE, ROWS_PER_TILE)
    pltpu.sync_copy(zacc_hbm.at[rslc], acc_s.at[rslc])
    if with_deg:
        pltpu.sync_copy(zdeg_hbm.at[rslc], deg_s.at[rslc])
        pltpu.sync_copy(ones_hbm, onesb)

    def i_start(k, b):
        off = ebase + k * CHUNK
        pltpu.make_async_copy(src_hbm.at[pl.ds(off, CHUNK)], idxs_v.at[b],
                              isem[b]).start()
        pltpu.make_async_copy(dst_hbm.at[pl.ds(off, CHUNK)], idxd_v.at[b],
                              isem[b]).start()

    def i_wait(b):
        pltpu.make_async_copy(src_hbm.at[pl.ds(0, CHUNK)], idxs_v.at[b],
                              isem[b]).wait()
        pltpu.make_async_copy(dst_hbm.at[pl.ds(0, CHUNK)], idxd_v.at[b],
                              isem[b]).wait()

    def g_start(b):
        pltpu.make_async_copy(h_hbm.at[idxs_v.at[b]], rowb.at[b],
                              gsem[b]).start()

    def g_wait(b):
        pltpu.make_async_copy(h_hbm.at[idxs_v.at[b]], rowb.at[b],
                              gsem[b]).wait()

    def s_start(b):
        pltpu.make_async_copy(rowb.at[b], acc_s.at[idxd_v.at[b]],
                              ssem[b]).start(add=True)

    def s_wait(b):
        pltpu.make_async_copy(rowb.at[b], acc_s.at[idxd_v.at[b]],
                              ssem[b]).wait()

    def d_start(b):
        pltpu.make_async_copy(onesb, deg_s.at[idxd_v.at[b]],
                              dsem[b]).start(add=True)

    def d_wait(b):
        pltpu.make_async_copy(onesb, deg_s.at[idxd_v.at[b]],
                              dsem[b]).wait()

    i_start(0, 0)
    i_start(1, 1)
    i_wait(0)
    g_start(0)
    plsc.subcore_barrier()

    @pl.loop(0, N_FULL // NBUF)
    def _outer(g):
        for b in range(NBUF):
            j = g * NBUF + b
            b1 = (b + 1) % NBUF
            b2 = (b + 2) % NBUF

            @pl.when(j >= 2)
            def _():
                s_wait(b2)
                if with_deg:
                    d_wait(b2)

            @pl.when(j < N_FULL - 2)
            def _():
                i_start(j + 2, b2)

            @pl.when(j < N_FULL - 1)
            def _():
                i_wait(b1)
                g_start(b1)

            g_wait(b)
            s_start(b)
            if with_deg:
                d_start(b)

    for j in (N_FULL - 2, N_FULL - 1):
        s_wait(j % NBUF)
        if with_deg:
            d_wait(j % NBUF)

    toff = ebase + N_FULL * CHUNK
    pltpu.sync_copy(src_hbm.at[pl.ds(toff, TAIL)], idxs_r)
    pltpu.sync_copy(dst_hbm.at[pl.ds(toff, TAIL)], idxd_r)
    pltpu.async_copy(h_hbm.at[idxs_r], rowr, gsem[0]).wait()
    pltpu.sync_copy(rowr, acc_s.at[idxd_r], add=True)
    if with_deg:
        pltpu.sync_copy(onesb.at[pl.ds(0, TAIL)], deg_s.at[idxd_r], add=True)

    plsc.subcore_barrier()

    @pl.when(c == 0)
    def _():
        pltpu.sync_copy(acc_s.at[rslc], acc0_hbm.at[rslc])
        if with_deg:
            pltpu.sync_copy(deg_s.at[rslc], deg0_hbm.at[rslc])

    @pl.when(c == 1)
    def _():
        pltpu.sync_copy(acc_s.at[rslc], acc1_hbm.at[rslc])
        if with_deg:
            pltpu.sync_copy(deg_s.at[rslc], deg1_hbm.at[rslc])


def _make_sc_layer(with_deg):
    mesh = plsc.VectorSubcoreMesh(core_axis_name="c", subcore_axis_name="s")
    f32 = jnp.float32
    i32 = jnp.int32
    out_type = [jax.ShapeDtypeStruct((N_PAD, DIM), f32),
                jax.ShapeDtypeStruct((N_PAD, DIM), f32)]
    scratch = [pltpu.VMEM_SHARED((N_PAD, DIM), f32)]
    if with_deg:
        out_type += [jax.ShapeDtypeStruct((N_PAD, DEG_W), f32),
                     jax.ShapeDtypeStruct((N_PAD, DEG_W), f32)]
        scratch += [pltpu.VMEM_SHARED((N_PAD, DEG_W), f32)]
    scratch += [pltpu.VMEM((NBUF, CHUNK), i32),
                pltpu.VMEM((NBUF, CHUNK), i32),
                pltpu.VMEM((NBUF, CHUNK, DIM), f32)]
    if with_deg:
        scratch += [pltpu.VMEM((CHUNK, DEG_W), f32)]
    scratch += [pltpu.VMEM((TAIL,), i32),
                pltpu.VMEM((TAIL,), i32),
                pltpu.VMEM((TAIL, DIM), f32)]
    nsem = 4 * NBUF if with_deg else 3 * NBUF
    scratch += [pltpu.SemaphoreType.DMA] * nsem
    return pl.kernel(functools.partial(_sc_layer_body, with_deg),
                     out_type=out_type, mesh=mesh, scratch_types=scratch,
                     compiler_params=pltpu.CompilerParams(
                         use_tc_tiling_on_sc=False),
                     name="sc_gnn_layer_deg" if with_deg else "sc_gnn_layer")


_sc_layer_with_deg = _make_sc_layer(True)
_sc_layer = _make_sc_layer(False)


def _tc_body(relu, acc0, acc1, deg0, deg1, w0, b0, w1, b1, out):
    inv0 = 1.0 / jnp.maximum(deg0[...][:, 0:1], 1.0)
    inv1 = 1.0 / jnp.maximum(deg1[...][:, 0:1], 1.0)
    y = (jnp.dot(acc0[...] * inv0, w0[...], preferred_element_type=jnp.float32)
         + jnp.dot(acc1[...] * inv1, w1[...], preferred_element_type=jnp.float32)
         + b0[...] + b1[...])
    if relu:
        y = jnp.maximum(y, 0.0)
    out[...] = y


_TC_ROWS = 1024


def _tc_layer(relu, acc0, acc1, deg0, deg1, w0, b0, w1, b1):
    grid = (N_PAD // _TC_ROWS,)
    rb = lambda i: (i, 0)
    fix = lambda i: (0, 0)
    return pl.pallas_call(
        functools.partial(_tc_body, relu),
        grid=grid,
        in_specs=[
            pl.BlockSpec((_TC_ROWS, DIM), rb),
            pl.BlockSpec((_TC_ROWS, DIM), rb),
            pl.BlockSpec((_TC_ROWS, DEG_W), rb),
            pl.BlockSpec((_TC_ROWS, DEG_W), rb),
            pl.BlockSpec((DIM, DIM), fix),
            pl.BlockSpec((1, DIM), fix),
            pl.BlockSpec((DIM, DIM), fix),
            pl.BlockSpec((1, DIM), fix),
        ],
        out_specs=pl.BlockSpec((_TC_ROWS, DIM), rb),
        out_shape=jax.ShapeDtypeStruct((N_PAD, DIM), jnp.float32),
    )(acc0, acc1, deg0, deg1, w0, b0, w1, b1)


def kernel(x, edge_index_rel0, edge_index_rel1, W0_0, b0_0, W0_1, b0_1,
           W1_0, b1_0, W1_1, b1_1, W2_0, b2_0, W2_1, b2_1):
    f32 = jnp.float32
    src = jnp.concatenate([edge_index_rel0[0], edge_index_rel1[0]])
    dst = jnp.concatenate([edge_index_rel0[1], edge_index_rel1[1]])
    zacc = jnp.zeros((N_PAD, DIM), f32)
    zdeg = jnp.zeros((N_PAD, DEG_W), f32)
    ones = jnp.ones((CHUNK, DEG_W), f32)

    acc0, acc1, deg0, deg1 = _sc_layer_with_deg(x, src, dst, zacc, zdeg, ones)
    h = _tc_layer(True, acc0, acc1, deg0, deg1,
                  W0_0, b0_0.reshape(1, DIM), W0_1, b0_1.reshape(1, DIM))

    acc0, acc1 = _sc_layer(h, src, dst, zacc)
    h = _tc_layer(True, acc0, acc1, deg0, deg1,
                  W1_0, b1_0.reshape(1, DIM), W1_1, b1_1.reshape(1, DIM))

    acc0, acc1 = _sc_layer(h, src, dst, zacc)
    h = _tc_layer(False, acc0, acc1, deg0, deg1,
                  W2_0, b2_0.reshape(1, DIM), W2_1, b2_1.reshape(1, DIM))
    return h[:N_NODES]


# speedup vs baseline: 10.8483x; 1.0519x over previous
import functools

import jax
import jax.numpy as jnp
from jax import lax
from jax.experimental import pallas as pl
from jax.experimental.pallas import tpu as pltpu
from jax.experimental.pallas import tpu_sc as plsc

N_NODES = 10000
N_PAD = 10240
E_PER_REL = 160000
DIM = 128

NC = 2
NS = 16
EDGES_PER_TILE = E_PER_REL // NS
ROWS_PER_TILE = N_PAD // NS
DEG_W = 16

CHUNK = 64
N_FULL = EDGES_PER_TILE // CHUNK
TAIL = EDGES_PER_TILE - N_FULL * CHUNK
NBUF = 4
IDXB = 8
UNROLL = 8
N_MAIN = (N_FULL // UNROLL) * UNROLL


def _sc_layer_body(with_deg, *refs):
    if with_deg:
        (h_hbm, src_hbm, dst_hbm, zacc_hbm, zdeg_hbm, ones_hbm,
         acc0_hbm, acc1_hbm, deg0_hbm, deg1_hbm,
         acc_s, deg_s, idxs_v, idxd_v, rowb, onesb,
         idxs_r, idxd_r, rowr, *sems) = refs
    else:
        (h_hbm, src_hbm, dst_hbm, zacc_hbm,
         acc0_hbm, acc1_hbm,
         acc_s, idxs_v, idxd_v, rowb,
         idxs_r, idxd_r, rowr, *sems) = refs
        deg_s = zdeg_hbm = ones_hbm = onesb = None
    gsem = sems[:NBUF]
    ssem = sems[NBUF:2 * NBUF]
    isem = sems[2 * NBUF:2 * NBUF + IDXB]
    dsem = sems[2 * NBUF + IDXB:]

    c = lax.axis_index("c")
    s = lax.axis_index("s")
    ebase = (c * NS + s) * EDGES_PER_TILE

    rslc = pl.ds(s * ROWS_PER_TILE, ROWS_PER_TILE)
    pltpu.sync_copy(zacc_hbm.at[rslc], acc_s.at[rslc])
    if with_deg:
        pltpu.sync_copy(zdeg_hbm.at[rslc], deg_s.at[rslc])
        pltpu.sync_copy(ones_hbm, onesb)

    def i_start(k, bi):
        off = ebase + k * CHUNK
        pltpu.make_async_copy(src_hbm.at[pl.ds(off, CHUNK)], idxs_v.at[bi],
                              isem[bi]).start()
        pltpu.make_async_copy(dst_hbm.at[pl.ds(off, CHUNK)], idxd_v.at[bi],
                              isem[bi]).start()

    def i_wait(bi):
        pltpu.make_async_copy(src_hbm.at[pl.ds(0, CHUNK)], idxs_v.at[bi],
                              isem[bi]).wait()
        pltpu.make_async_copy(dst_hbm.at[pl.ds(0, CHUNK)], idxd_v.at[bi],
                              isem[bi]).wait()

    def g_start(b, bi):
        pltpu.make_async_copy(h_hbm.at[idxs_v.at[bi]], rowb.at[b],
                              gsem[b]).start()

    def g_wait(b):
        pltpu.make_async_copy(h_hbm.at[idxs_v.at[0]], rowb.at[b],
                              gsem[b]).wait()

    def s_start(b, bi):
        pltpu.make_async_copy(rowb.at[b], acc_s.at[idxd_v.at[bi]],
                              ssem[b]).start(add=True)

    def s_wait(b):
        pltpu.make_async_copy(rowb.at[b], acc_s.at[idxd_v.at[0]],
                              ssem[b]).wait()

    def d_start(b, bi):
        pltpu.make_async_copy(onesb, deg_s.at[idxd_v.at[bi]],
                              dsem[b]).start(add=True)

    def d_wait(b):
        pltpu.make_async_copy(onesb, deg_s.at[idxd_v.at[0]],
                              dsem[b]).wait()

    for k in range(3):
        i_start(k, k)
    i_wait(0)
    g_start(0, 0)
    i_wait(1)
    g_start(1, 1)
    plsc.subcore_barrier()

    @pl.loop(0, N_MAIN // UNROLL)
    def _outer(g):
        j0 = g * UNROLL
        for b in range(UNROLL):
            j = j0 + b
            @pl.when(j >= 2)
            def _():
                s_wait((b + 2) % NBUF)
                if with_deg:
                    d_wait((b + 2) % NBUF)

            i_start(j + 3, (b + 3) % IDXB)
            i_wait((b + 2) % IDXB)
            g_start((b + 2) % NBUF, (b + 2) % IDXB)
            g_wait(b % NBUF)
            s_start(b % NBUF, b % IDXB)
            if with_deg:
                d_start(b % NBUF, b % IDXB)

    for j in range(N_MAIN, N_FULL):
        b = j % NBUF
        bi = j % IDXB
        s_wait((b + 2) % NBUF)
        if with_deg:
            d_wait((b + 2) % NBUF)
        if j < N_FULL - 3:
            i_start(j + 3, (bi + 3) % IDXB)
        if j < N_FULL - 2:
            i_wait((bi + 2) % IDXB)
            g_start((b + 2) % NBUF, (bi + 2) % IDXB)
        g_wait(b)
        s_start(b, bi)
        if with_deg:
            d_start(b, bi)

    for j in (N_FULL - 2, N_FULL - 1):
        s_wait(j % NBUF)
        if with_deg:
            d_wait(j % NBUF)

    toff = ebase + N_FULL * CHUNK
    pltpu.sync_copy(src_hbm.at[pl.ds(toff, TAIL)], idxs_r)
    pltpu.sync_copy(dst_hbm.at[pl.ds(toff, TAIL)], idxd_r)
    pltpu.async_copy(h_hbm.at[idxs_r], rowr, gsem[0]).wait()
    pltpu.sync_copy(rowr, acc_s.at[idxd_r], add=True)
    if with_deg:
        pltpu.sync_copy(onesb.at[pl.ds(0, TAIL)], deg_s.at[idxd_r], add=True)

    plsc.subcore_barrier()

    @pl.when(c == 0)
    def _():
        pltpu.sync_copy(acc_s.at[rslc], acc0_hbm.at[rslc])
        if with_deg:
            pltpu.sync_copy(deg_s.at[rslc], deg0_hbm.at[rslc])

    @pl.when(c == 1)
    def _():
        pltpu.sync_copy(acc_s.at[rslc], acc1_hbm.at[rslc])
        if with_deg:
            pltpu.sync_copy(deg_s.at[rslc], deg1_hbm.at[rslc])


def _make_sc_layer(with_deg):
    mesh = plsc.VectorSubcoreMesh(core_axis_name="c", subcore_axis_name="s")
    f32 = jnp.float32
    i32 = jnp.int32
    out_type = [jax.ShapeDtypeStruct((N_PAD, DIM), f32),
                jax.ShapeDtypeStruct((N_PAD, DIM), f32)]
    scratch = [pltpu.VMEM_SHARED((N_PAD, DIM), f32)]
    if with_deg:
        out_type += [jax.ShapeDtypeStruct((N_PAD, DEG_W), f32),
                     jax.ShapeDtypeStruct((N_PAD, DEG_W), f32)]
        scratch += [pltpu.VMEM_SHARED((N_PAD, DEG_W), f32)]
    scratch += [pltpu.VMEM((IDXB, CHUNK), i32),
                pltpu.VMEM((IDXB, CHUNK), i32),
                pltpu.VMEM((NBUF, CHUNK, DIM), f32)]
    if with_deg:
        scratch += [pltpu.VMEM((CHUNK, DEG_W), f32)]
    scratch += [pltpu.VMEM((TAIL,), i32),
                pltpu.VMEM((TAIL,), i32),
                pltpu.VMEM((TAIL, DIM), f32)]
    nsem = (3 * NBUF if with_deg else 2 * NBUF) + IDXB
    scratch += [pltpu.SemaphoreType.DMA] * nsem
    return pl.kernel(functools.partial(_sc_layer_body, with_deg),
                     out_type=out_type, mesh=mesh, scratch_types=scratch,
                     compiler_params=pltpu.CompilerParams(
                         use_tc_tiling_on_sc=False),
                     name="sc_gnn_layer_deg" if with_deg else "sc_gnn_layer")


_sc_layer_with_deg = _make_sc_layer(True)
_sc_layer = _make_sc_layer(False)


def _tc_body(relu, acc0, acc1, deg0, deg1, w0, b0, w1, b1, out):
    inv0 = 1.0 / jnp.maximum(deg0[...][:, 0:1], 1.0)
    inv1 = 1.0 / jnp.maximum(deg1[...][:, 0:1], 1.0)
    y = (jnp.dot(acc0[...] * inv0, w0[...], preferred_element_type=jnp.float32)
         + jnp.dot(acc1[...] * inv1, w1[...], preferred_element_type=jnp.float32)
         + b0[...] + b1[...])
    if relu:
        y = jnp.maximum(y, 0.0)
    out[...] = y


_TC_ROWS = 1024


def _tc_layer(relu, acc0, acc1, deg0, deg1, w0, b0, w1, b1):
    grid = (N_PAD // _TC_ROWS,)
    rb = lambda i: (i, 0)
    fix = lambda i: (0, 0)
    return pl.pallas_call(
        functools.partial(_tc_body, relu),
        grid=grid,
        in_specs=[
            pl.BlockSpec((_TC_ROWS, DIM), rb),
            pl.BlockSpec((_TC_ROWS, DIM), rb),
            pl.BlockSpec((_TC_ROWS, DEG_W), rb),
            pl.BlockSpec((_TC_ROWS, DEG_W), rb),
            pl.BlockSpec((DIM, DIM), fix),
            pl.BlockSpec((1, DIM), fix),
            pl.BlockSpec((DIM, DIM), fix),
            pl.BlockSpec((1, DIM), fix),
        ],
        out_specs=pl.BlockSpec((_TC_ROWS, DIM), rb),
        out_shape=jax.ShapeDtypeStruct((N_PAD, DIM), jnp.float32),
    )(acc0, acc1, deg0, deg1, w0, b0, w1, b1)


def kernel(x, edge_index_rel0, edge_index_rel1, W0_0, b0_0, W0_1, b0_1,
           W1_0, b1_0, W1_1, b1_1, W2_0, b2_0, W2_1, b2_1):
    f32 = jnp.float32
    src = jnp.concatenate([edge_index_rel0[0], edge_index_rel1[0]])
    dst = jnp.concatenate([edge_index_rel0[1], edge_index_rel1[1]])
    zacc = jnp.zeros((N_PAD, DIM), f32)
    zdeg = jnp.zeros((N_PAD, DEG_W), f32)
    ones = jnp.ones((CHUNK, DEG_W), f32)

    acc0, acc1, deg0, deg1 = _sc_layer_with_deg(x, src, dst, zacc, zdeg, ones)
    h = _tc_layer(True, acc0, acc1, deg0, deg1,
                  W0_0, b0_0.reshape(1, DIM), W0_1, b0_1.reshape(1, DIM))

    acc0, acc1 = _sc_layer(h, src, dst, zacc)
    h = _tc_layer(True, acc0, acc1, deg0, deg1,
                  W1_0, b1_0.reshape(1, DIM), W1_1, b1_1.reshape(1, DIM))

    acc0, acc1 = _sc_layer(h, src, dst, zacc)
    h = _tc_layer(False, acc0, acc1, deg0, deg1,
                  W2_0, b2_0.reshape(1, DIM), W2_1, b2_1.reshape(1, DIM))
    return h[:N_NODES]


# speedup vs baseline: 10.9515x; 1.0095x over previous
import functools

import jax
import jax.numpy as jnp
from jax import lax
from jax.experimental import pallas as pl
from jax.experimental.pallas import tpu as pltpu
from jax.experimental.pallas import tpu_sc as plsc

N_NODES = 10000
N_PAD = 10240
E_PER_REL = 160000
DIM = 128

NC = 2
NS = 16
EDGES_PER_TILE = E_PER_REL // NS
ROWS_PER_TILE = N_PAD // NS
DEG_W = 16

CHUNK = 64
N_FULL = EDGES_PER_TILE // CHUNK
TAIL = EDGES_PER_TILE - N_FULL * CHUNK
NBUF = 4
IDXB = 8
UNROLL = 8
N_MAIN = (N_FULL // UNROLL) * UNROLL


def _sc_layer_body(with_deg, *refs):
    if with_deg:
        (h_hbm, src_hbm, dst_hbm, zacc_hbm, zdeg_hbm, ones_hbm,
         acc0_hbm, acc1_hbm, deg0_hbm, deg1_hbm,
         acc_s, deg_s, idxs_v, idxd_v, rowb, onesb,
         idxs_r, idxd_r, rowr, *sems) = refs
    else:
        (h_hbm, src_hbm, dst_hbm, zacc_hbm,
         acc0_hbm, acc1_hbm,
         acc_s, idxs_v, idxd_v, rowb,
         idxs_r, idxd_r, rowr, *sems) = refs
        deg_s = zdeg_hbm = ones_hbm = onesb = None
    gsem = sems[:NBUF]
    ssem = sems[NBUF:2 * NBUF]
    isem = sems[2 * NBUF:2 * NBUF + IDXB]
    zsem = sems[2 * NBUF + IDXB]
    dsem = sems[2 * NBUF + IDXB + 1:]

    c = lax.axis_index("c")
    s = lax.axis_index("s")
    ebase = (c * NS + s) * EDGES_PER_TILE

    rslc = pl.ds(s * ROWS_PER_TILE, ROWS_PER_TILE)
    zdescs = [pltpu.make_async_copy(zacc_hbm.at[rslc], acc_s.at[rslc], zsem)]
    if with_deg:
        zdescs.append(pltpu.make_async_copy(zdeg_hbm.at[rslc],
                                            deg_s.at[rslc], zsem))
        zdescs.append(pltpu.make_async_copy(ones_hbm, onesb, zsem))
    for zd in zdescs:
        zd.start()

    def i_start(k, bi):
        off = ebase + k * CHUNK
        pltpu.make_async_copy(src_hbm.at[pl.ds(off, CHUNK)], idxs_v.at[bi],
                              isem[bi]).start()
        pltpu.make_async_copy(dst_hbm.at[pl.ds(off, CHUNK)], idxd_v.at[bi],
                              isem[bi]).start()

    def i_wait(bi):
        pltpu.make_async_copy(src_hbm.at[pl.ds(0, CHUNK)], idxs_v.at[bi],
                              isem[bi]).wait()
        pltpu.make_async_copy(dst_hbm.at[pl.ds(0, CHUNK)], idxd_v.at[bi],
                              isem[bi]).wait()

    def g_start(b, bi):
        pltpu.make_async_copy(h_hbm.at[idxs_v.at[bi]], rowb.at[b],
                              gsem[b]).start()

    def g_wait(b):
        pltpu.make_async_copy(h_hbm.at[idxs_v.at[0]], rowb.at[b],
                              gsem[b]).wait()

    def s_start(b, bi):
        pltpu.make_async_copy(rowb.at[b], acc_s.at[idxd_v.at[bi]],
                              ssem[b]).start(add=True)

    def s_wait(b):
        pltpu.make_async_copy(rowb.at[b], acc_s.at[idxd_v.at[0]],
                              ssem[b]).wait()

    def d_start(b, bi):
        pltpu.make_async_copy(onesb, deg_s.at[idxd_v.at[bi]],
                              dsem[b]).start(add=True)

    def d_wait(b):
        pltpu.make_async_copy(onesb, deg_s.at[idxd_v.at[0]],
                              dsem[b]).wait()

    for k in range(3):
        i_start(k, k)
    i_wait(0)
    g_start(0, 0)
    i_wait(1)
    g_start(1, 1)
    for zd in zdescs:
        zd.wait()
    plsc.subcore_barrier()

    @pl.loop(0, N_MAIN // UNROLL)
    def _outer(g):
        j0 = g * UNROLL
        for b in range(UNROLL):
            j = j0 + b
            @pl.when(j >= 2)
            def _():
                s_wait((b + 2) % NBUF)
                if with_deg:
                    d_wait((b + 2) % NBUF)

            i_start(j + 3, (b + 3) % IDXB)
            i_wait((b + 2) % IDXB)
            g_start((b + 2) % NBUF, (b + 2) % IDXB)
            g_wait(b % NBUF)
            s_start(b % NBUF, b % IDXB)
            if with_deg:
                d_start(b % NBUF, b % IDXB)

    for j in range(N_MAIN, N_FULL):
        b = j % NBUF
        bi = j % IDXB
        s_wait((b + 2) % NBUF)
        if with_deg:
            d_wait((b + 2) % NBUF)
        if j < N_FULL - 3:
            i_start(j + 3, (bi + 3) % IDXB)
        if j < N_FULL - 2:
            i_wait((bi + 2) % IDXB)
            g_start((b + 2) % NBUF, (bi + 2) % IDXB)
        g_wait(b)
        s_start(b, bi)
        if with_deg:
            d_start(b, bi)

    for j in (N_FULL - 2, N_FULL - 1):
        s_wait(j % NBUF)
        if with_deg:
            d_wait(j % NBUF)

    toff = ebase + N_FULL * CHUNK
    pltpu.sync_copy(src_hbm.at[pl.ds(toff, TAIL)], idxs_r)
    pltpu.sync_copy(dst_hbm.at[pl.ds(toff, TAIL)], idxd_r)
    pltpu.async_copy(h_hbm.at[idxs_r], rowr, gsem[0]).wait()
    pltpu.sync_copy(rowr, acc_s.at[idxd_r], add=True)
    if with_deg:
        pltpu.sync_copy(onesb.at[pl.ds(0, TAIL)], deg_s.at[idxd_r], add=True)

    plsc.subcore_barrier()

    @pl.when(c == 0)
    def _():
        pltpu.sync_copy(acc_s.at[rslc], acc0_hbm.at[rslc])
        if with_deg:
            pltpu.sync_copy(deg_s.at[rslc], deg0_hbm.at[rslc])

    @pl.when(c == 1)
    def _():
        pltpu.sync_copy(acc_s.at[rslc], acc1_hbm.at[rslc])
        if with_deg:
            pltpu.sync_copy(deg_s.at[rslc], deg1_hbm.at[rslc])


def _make_sc_layer(with_deg):
    mesh = plsc.VectorSubcoreMesh(core_axis_name="c", subcore_axis_name="s")
    f32 = jnp.float32
    i32 = jnp.int32
    out_type = [jax.ShapeDtypeStruct((N_PAD, DIM), f32),
                jax.ShapeDtypeStruct((N_PAD, DIM), f32)]
    scratch = [pltpu.VMEM_SHARED((N_PAD, DIM), f32)]
    if with_deg:
        out_type += [jax.ShapeDtypeStruct((N_PAD, DEG_W), f32),
                     jax.ShapeDtypeStruct((N_PAD, DEG_W), f32)]
        scratch += [pltpu.VMEM_SHARED((N_PAD, DEG_W), f32)]
    scratch += [pltpu.VMEM((IDXB, CHUNK), i32),
                pltpu.VMEM((IDXB, CHUNK), i32),
                pltpu.VMEM((NBUF, CHUNK, DIM), f32)]
    if with_deg:
        scratch += [pltpu.VMEM((CHUNK, DEG_W), f32)]
    scratch += [pltpu.VMEM((TAIL,), i32),
                pltpu.VMEM((TAIL,), i32),
                pltpu.VMEM((TAIL, DIM), f32)]
    nsem = (3 * NBUF if with_deg else 2 * NBUF) + IDXB + 1
    scratch += [pltpu.SemaphoreType.DMA] * nsem
    return pl.kernel(functools.partial(_sc_layer_body, with_deg),
                     out_type=out_type, mesh=mesh, scratch_types=scratch,
                     compiler_params=pltpu.CompilerParams(
                         use_tc_tiling_on_sc=False),
                     name="sc_gnn_layer_deg" if with_deg else "sc_gnn_layer")


_sc_layer_with_deg = _make_sc_layer(True)
_sc_layer = _make_sc_layer(False)


def _tc_body(relu, acc0, acc1, deg0, deg1, w0, b0, w1, b1, out):
    inv0 = 1.0 / jnp.maximum(deg0[...][:, 0:1], 1.0)
    inv1 = 1.0 / jnp.maximum(deg1[...][:, 0:1], 1.0)
    y = (jnp.dot(acc0[...] * inv0, w0[...], preferred_element_type=jnp.float32)
         + jnp.dot(acc1[...] * inv1, w1[...], preferred_element_type=jnp.float32)
         + b0[...] + b1[...])
    if relu:
        y = jnp.maximum(y, 0.0)
    out[...] = y


_TC_ROWS = 1024


def _tc_layer(relu, acc0, acc1, deg0, deg1, w0, b0, w1, b1):
    grid = (N_PAD // _TC_ROWS,)
    rb = lambda i: (i, 0)
    fix = lambda i: (0, 0)
    return pl.pallas_call(
        functools.partial(_tc_body, relu),
        grid=grid,
        in_specs=[
            pl.BlockSpec((_TC_ROWS, DIM), rb),
            pl.BlockSpec((_TC_ROWS, DIM), rb),
            pl.BlockSpec((_TC_ROWS, DEG_W), rb),
            pl.BlockSpec((_TC_ROWS, DEG_W), rb),
            pl.BlockSpec((DIM, DIM), fix),
            pl.BlockSpec((1, DIM), fix),
            pl.BlockSpec((DIM, DIM), fix),
            pl.BlockSpec((1, DIM), fix),
        ],
        out_specs=pl.BlockSpec((_TC_ROWS, DIM), rb),
        out_shape=jax.ShapeDtypeStruct((N_PAD, DIM), jnp.float32),
    )(acc0, acc1, deg0, deg1, w0, b0, w1, b1)


def kernel(x, edge_index_rel0, edge_index_rel1, W0_0, b0_0, W0_1, b0_1,
           W1_0, b1_0, W1_1, b1_1, W2_0, b2_0, W2_1, b2_1):
    f32 = jnp.float32
    src = jnp.concatenate([edge_index_rel0[0], edge_index_rel1[0]])
    dst = jnp.concatenate([edge_index_rel0[1], edge_index_rel1[1]])
    zacc = jnp.zeros((N_PAD, DIM), f32)
    zdeg = jnp.zeros((N_PAD, DEG_W), f32)
    ones = jnp.ones((CHUNK, DEG_W), f32)

    acc0, acc1, deg0, deg1 = _sc_layer_with_deg(x, src, dst, zacc, zdeg, ones)
    h = _tc_layer(True, acc0, acc1, deg0, deg1,
                  W0_0, b0_0.reshape(1, DIM), W0_1, b0_1.reshape(1, DIM))

    acc0, acc1 = _sc_layer(h, src, dst, zacc)
    h = _tc_layer(True, acc0, acc1, deg0, deg1,
                  W1_0, b1_0.reshape(1, DIM), W1_1, b1_1.reshape(1, DIM))

    acc0, acc1 = _sc_layer(h, src, dst, zacc)
    h = _tc_layer(False, acc0, acc1, deg0, deg1,
                  W2_0, b2_0.reshape(1, DIM), W2_1, b2_1.reshape(1, DIM))
    return h[:N_NODES]


# speedup vs baseline: 11.0681x; 1.0106x over previous
import functools

import jax
import jax.numpy as jnp
from jax import lax
from jax.experimental import pallas as pl
from jax.experimental.pallas import tpu as pltpu
from jax.experimental.pallas import tpu_sc as plsc

N_NODES = 10000
N_PAD = 10240
E_PER_REL = 160000
DIM = 128

NC = 2
NS = 16
EDGES_PER_TILE = E_PER_REL // NS
ROWS_PER_TILE = N_PAD // NS
DEG_W = 16

CHUNK = 64
N_FULL = EDGES_PER_TILE // CHUNK
TAIL = EDGES_PER_TILE - N_FULL * CHUNK
NBUF = 4
IDXB = 8
UNROLL = 8
N_MAIN = (N_FULL // UNROLL) * UNROLL


def _sc_layer_body(with_deg, *refs):
    if with_deg:
        (h_hbm, src_hbm, dst_hbm, zacc_hbm, zdeg_hbm, ones_hbm,
         acc0_hbm, acc1_hbm, deg0_hbm, deg1_hbm,
         acc_s, deg_s, idxs_v, idxd_v, rowb, onesb,
         idxs_r, idxd_r, rowr, *sems) = refs
    else:
        (h_hbm, src_hbm, dst_hbm, zacc_hbm,
         acc0_hbm, acc1_hbm,
         acc_s, idxs_v, idxd_v, rowb,
         idxs_r, idxd_r, rowr, *sems) = refs
        deg_s = zdeg_hbm = ones_hbm = onesb = None
    gsem = sems[:NBUF]
    ssem = sems[NBUF:2 * NBUF]
    isem = sems[2 * NBUF:2 * NBUF + IDXB]
    zsem = sems[2 * NBUF + IDXB]
    dsem = sems[2 * NBUF + IDXB + 1:]

    c = lax.axis_index("c")
    s = lax.axis_index("s")
    ebase = (c * NS + s) * EDGES_PER_TILE

    rslc = pl.ds(s * ROWS_PER_TILE, ROWS_PER_TILE)
    zdescs = [pltpu.make_async_copy(zacc_hbm.at[rslc], acc_s.at[rslc], zsem)]
    if with_deg:
        zdescs.append(pltpu.make_async_copy(zdeg_hbm.at[rslc],
                                            deg_s.at[rslc], zsem))
        zdescs.append(pltpu.make_async_copy(ones_hbm, onesb, zsem))
    for zd in zdescs:
        zd.start()

    def i_start(k, bi):
        off = ebase + k * CHUNK
        pltpu.make_async_copy(src_hbm.at[pl.ds(off, CHUNK)], idxs_v.at[bi],
                              isem[bi]).start()
        pltpu.make_async_copy(dst_hbm.at[pl.ds(off, CHUNK)], idxd_v.at[bi],
                              isem[bi]).start()

    def i_wait(bi):
        pltpu.make_async_copy(src_hbm.at[pl.ds(0, CHUNK)], idxs_v.at[bi],
                              isem[bi]).wait()
        pltpu.make_async_copy(dst_hbm.at[pl.ds(0, CHUNK)], idxd_v.at[bi],
                              isem[bi]).wait()

    def g_start(b, bi):
        pltpu.make_async_copy(h_hbm.at[idxs_v.at[bi]], rowb.at[b],
                              gsem[b]).start()

    def g_wait(b):
        pltpu.make_async_copy(h_hbm.at[idxs_v.at[0]], rowb.at[b],
                              gsem[b]).wait()

    def s_start(b, bi):
        pltpu.make_async_copy(rowb.at[b], acc_s.at[idxd_v.at[bi]],
                              ssem[b]).start(add=True)

    def s_wait(b):
        pltpu.make_async_copy(rowb.at[b], acc_s.at[idxd_v.at[0]],
                              ssem[b]).wait()

    def d_start(b, bi):
        pltpu.make_async_copy(onesb, deg_s.at[idxd_v.at[bi]],
                              dsem[b]).start(add=True)

    def d_wait(b):
        pltpu.make_async_copy(onesb, deg_s.at[idxd_v.at[0]],
                              dsem[b]).wait()

    for k in range(3):
        i_start(k, k)
    i_wait(0)
    g_start(0, 0)
    i_wait(1)
    g_start(1, 1)
    for zd in zdescs:
        zd.wait()
    plsc.subcore_barrier()

    @pl.loop(0, N_MAIN // UNROLL)
    def _outer(g):
        j0 = g * UNROLL
        for b in range(UNROLL):
            j = j0 + b
            @pl.when(j >= 2)
            def _():
                s_wait((b + 2) % NBUF)
                if with_deg:
                    d_wait((b + 2) % NBUF)

            i_start(j + 3, (b + 3) % IDXB)
            i_wait((b + 2) % IDXB)
            g_start((b + 2) % NBUF, (b + 2) % IDXB)
            g_wait(b % NBUF)
            s_start(b % NBUF, b % IDXB)
            if with_deg:
                d_start(b % NBUF, b % IDXB)

    for j in range(N_MAIN, N_FULL):
        b = j % NBUF
        bi = j % IDXB
        s_wait((b + 2) % NBUF)
        if with_deg:
            d_wait((b + 2) % NBUF)
        if j < N_FULL - 3:
            i_start(j + 3, (bi + 3) % IDXB)
        if j < N_FULL - 2:
            i_wait((bi + 2) % IDXB)
            g_start((b + 2) % NBUF, (bi + 2) % IDXB)
        g_wait(b)
        s_start(b, bi)
        if with_deg:
            d_start(b, bi)

    for j in (N_FULL - 2, N_FULL - 1):
        s_wait(j % NBUF)
        if with_deg:
            d_wait(j % NBUF)

    toff = ebase + N_FULL * CHUNK
    pltpu.sync_copy(src_hbm.at[pl.ds(toff, TAIL)], idxs_r)
    pltpu.sync_copy(dst_hbm.at[pl.ds(toff, TAIL)], idxd_r)
    pltpu.async_copy(h_hbm.at[idxs_r], rowr, gsem[0]).wait()
    pltpu.sync_copy(rowr, acc_s.at[idxd_r], add=True)
    if with_deg:
        pltpu.sync_copy(onesb.at[pl.ds(0, TAIL)], deg_s.at[idxd_r], add=True)

    plsc.subcore_barrier()

    @pl.when(c == 0)
    def _():
        pltpu.sync_copy(acc_s.at[rslc], acc0_hbm.at[rslc])
        if with_deg:
            pltpu.sync_copy(deg_s.at[rslc], deg0_hbm.at[rslc])

    @pl.when(c == 1)
    def _():
        pltpu.sync_copy(acc_s.at[rslc], acc1_hbm.at[rslc])
        if with_deg:
            pltpu.sync_copy(deg_s.at[rslc], deg1_hbm.at[rslc])


def _make_sc_layer(with_deg):
    mesh = plsc.VectorSubcoreMesh(core_axis_name="c", subcore_axis_name="s")
    f32 = jnp.float32
    i32 = jnp.int32
    out_type = [jax.ShapeDtypeStruct((N_PAD, DIM), f32),
                jax.ShapeDtypeStruct((N_PAD, DIM), f32)]
    scratch = [pltpu.VMEM_SHARED((N_PAD, DIM), f32)]
    if with_deg:
        out_type += [jax.ShapeDtypeStruct((N_PAD, DEG_W), f32),
                     jax.ShapeDtypeStruct((N_PAD, DEG_W), f32)]
        scratch += [pltpu.VMEM_SHARED((N_PAD, DEG_W), f32)]
    scratch += [pltpu.VMEM((IDXB, CHUNK), i32),
                pltpu.VMEM((IDXB, CHUNK), i32),
                pltpu.VMEM((NBUF, CHUNK, DIM), f32)]
    if with_deg:
        scratch += [pltpu.VMEM((CHUNK, DEG_W), f32)]
    scratch += [pltpu.VMEM((TAIL,), i32),
                pltpu.VMEM((TAIL,), i32),
                pltpu.VMEM((TAIL, DIM), f32)]
    nsem = (3 * NBUF if with_deg else 2 * NBUF) + IDXB + 1
    scratch += [pltpu.SemaphoreType.DMA] * nsem
    return pl.kernel(functools.partial(_sc_layer_body, with_deg),
                     out_type=out_type, mesh=mesh, scratch_types=scratch,
                     compiler_params=pltpu.CompilerParams(
                         use_tc_tiling_on_sc=False),
                     name="sc_gnn_layer_deg" if with_deg else "sc_gnn_layer")


_sc_layer_with_deg = _make_sc_layer(True)
_sc_layer = _make_sc_layer(False)


def _tc_body(relu, acc0, acc1, deg0, deg1, w0, b0, w1, b1, out):
    inv0 = 1.0 / jnp.maximum(deg0[...][:, 0:1], 1.0)
    inv1 = 1.0 / jnp.maximum(deg1[...][:, 0:1], 1.0)
    y = (jnp.dot(acc0[...] * inv0, w0[...], preferred_element_type=jnp.float32)
         + jnp.dot(acc1[...] * inv1, w1[...], preferred_element_type=jnp.float32)
         + b0[...] + b1[...])
    if relu:
        y = jnp.maximum(y, 0.0)
    out[...] = y


_TC_ROWS = 1000


def _tc_layer(relu, acc0, acc1, deg0, deg1, w0, b0, w1, b1):
    grid = (N_NODES // _TC_ROWS,)
    rb = lambda i: (i, 0)
    fix = lambda i: (0, 0)
    return pl.pallas_call(
        functools.partial(_tc_body, relu),
        grid=grid,
        in_specs=[
            pl.BlockSpec((_TC_ROWS, DIM), rb),
            pl.BlockSpec((_TC_ROWS, DIM), rb),
            pl.BlockSpec((_TC_ROWS, DEG_W), rb),
            pl.BlockSpec((_TC_ROWS, DEG_W), rb),
            pl.BlockSpec((DIM, DIM), fix),
            pl.BlockSpec((1, DIM), fix),
            pl.BlockSpec((DIM, DIM), fix),
            pl.BlockSpec((1, DIM), fix),
        ],
        out_specs=pl.BlockSpec((_TC_ROWS, DIM), rb),
        out_shape=jax.ShapeDtypeStruct((N_NODES, DIM), jnp.float32),
    )(acc0, acc1, deg0, deg1, w0, b0, w1, b1)


def kernel(x, edge_index_rel0, edge_index_rel1, W0_0, b0_0, W0_1, b0_1,
           W1_0, b1_0, W1_1, b1_1, W2_0, b2_0, W2_1, b2_1):
    f32 = jnp.float32
    src = jnp.concatenate([edge_index_rel0[0], edge_index_rel1[0]])
    dst = jnp.concatenate([edge_index_rel0[1], edge_index_rel1[1]])
    zacc = jnp.zeros((N_PAD, DIM), f32)
    zdeg = jnp.zeros((N_PAD, DEG_W), f32)
    ones = jnp.ones((CHUNK, DEG_W), f32)

    acc0, acc1, deg0, deg1 = _sc_layer_with_deg(x, src, dst, zacc, zdeg, ones)
    h = _tc_layer(True, acc0, acc1, deg0, deg1,
                  W0_0, b0_0.reshape(1, DIM), W0_1, b0_1.reshape(1, DIM))

    acc0, acc1 = _sc_layer(h, src, dst, zacc)
    h = _tc_layer(True, acc0, acc1, deg0, deg1,
                  W1_0, b1_0.reshape(1, DIM), W1_1, b1_1.reshape(1, DIM))

    acc0, acc1 = _sc_layer(h, src, dst, zacc)
    h = _tc_layer(False, acc0, acc1, deg0, deg1,
                  W2_0, b2_0.reshape(1, DIM), W2_1, b2_1.reshape(1, DIM))
    return h


# speedup vs baseline: 11.1898x; 1.0110x over previous
import functools

import jax
import jax.numpy as jnp
from jax import lax
from jax.experimental import pallas as pl
from jax.experimental.pallas import tpu as pltpu
from jax.experimental.pallas import tpu_sc as plsc

N_NODES = 10000
N_PAD = 10240
E_PER_REL = 160000
DIM = 128

NC = 2
NS = 16
EDGES_PER_TILE = E_PER_REL // NS
ROWS_PER_TILE = N_PAD // NS
DEG_W = 16

CHUNK = 64
N_FULL = EDGES_PER_TILE // CHUNK
TAIL = EDGES_PER_TILE - N_FULL * CHUNK
NBUF = 4
IDXB = 8
UNROLL = 8
N_MAIN = (N_FULL // UNROLL) * UNROLL


def _sc_layer_body(with_deg, *refs):
    if with_deg:
        (h_hbm, src_hbm, dst_hbm, zacc_hbm, zdeg_hbm, ones_hbm,
         acc0_hbm, acc1_hbm, deg0_hbm, deg1_hbm,
         acc_s, deg_s, idxs_v, idxd_v, rowb, onesb,
         idxs_r, idxd_r, rowr, *sems) = refs
    else:
        (h_hbm, src_hbm, dst_hbm, zacc_hbm,
         acc0_hbm, acc1_hbm,
         acc_s, idxs_v, idxd_v, rowb,
         idxs_r, idxd_r, rowr, *sems) = refs
        deg_s = zdeg_hbm = ones_hbm = onesb = None
    gsem = sems[:NBUF]
    ssem = sems[NBUF:2 * NBUF]
    isem = sems[2 * NBUF:2 * NBUF + IDXB]
    zsem = sems[2 * NBUF + IDXB]
    dsem = sems[2 * NBUF + IDXB + 1:]

    c = lax.axis_index("c")
    s = lax.axis_index("s")
    ebase = (c * NS + s) * EDGES_PER_TILE

    rslc = pl.ds(s * ROWS_PER_TILE, ROWS_PER_TILE)
    zdescs = [pltpu.make_async_copy(zacc_hbm.at[rslc], acc_s.at[rslc], zsem)]
    if with_deg:
        zdescs.append(pltpu.make_async_copy(zdeg_hbm.at[rslc],
                                            deg_s.at[rslc], zsem))
        zdescs.append(pltpu.make_async_copy(ones_hbm, onesb, zsem))
    for zd in zdescs:
        zd.start()

    def i_start(k, bi):
        off = ebase + k * CHUNK
        pltpu.make_async_copy(src_hbm.at[pl.ds(off, CHUNK)], idxs_v.at[bi],
                              isem[bi]).start()
        pltpu.make_async_copy(dst_hbm.at[pl.ds(off, CHUNK)], idxd_v.at[bi],
                              isem[bi]).start()

    def i_wait(bi):
        pltpu.make_async_copy(src_hbm.at[pl.ds(0, CHUNK)], idxs_v.at[bi],
                              isem[bi]).wait()
        pltpu.make_async_copy(dst_hbm.at[pl.ds(0, CHUNK)], idxd_v.at[bi],
                              isem[bi]).wait()

    def g_start(b, bi):
        pltpu.make_async_copy(h_hbm.at[idxs_v.at[bi]], rowb.at[b],
                              gsem[b]).start()

    def g_wait(b):
        pltpu.make_async_copy(h_hbm.at[idxs_v.at[0]], rowb.at[b],
                              gsem[b]).wait()

    def s_start(b, bi):
        pltpu.make_async_copy(rowb.at[b], acc_s.at[idxd_v.at[bi]],
                              ssem[b]).start(add=True)

    def s_wait(b):
        pltpu.make_async_copy(rowb.at[b], acc_s.at[idxd_v.at[0]],
                              ssem[b]).wait()

    def d_start(b, bi):
        pltpu.make_async_copy(onesb, deg_s.at[idxd_v.at[bi]],
                              dsem[b]).start(add=True)

    def d_wait(b):
        pltpu.make_async_copy(onesb, deg_s.at[idxd_v.at[0]],
                              dsem[b]).wait()

    toff = ebase + N_FULL * CHUNK
    t_is = pltpu.make_async_copy(src_hbm.at[pl.ds(toff, TAIL)], idxs_r, zsem)
    t_id = pltpu.make_async_copy(dst_hbm.at[pl.ds(toff, TAIL)], idxd_r, zsem)
    t_g = pltpu.make_async_copy(h_hbm.at[idxs_r], rowr, zsem)
    t_is.start()
    t_id.start()
    for k in range(3):
        i_start(k, k)
    i_wait(0)
    g_start(0, 0)
    i_wait(1)
    g_start(1, 1)
    for zd in zdescs:
        zd.wait()
    plsc.subcore_barrier()

    @pl.loop(0, N_MAIN // UNROLL)
    def _outer(g):
        j0 = g * UNROLL
        for b in range(UNROLL):
            j = j0 + b
            @pl.when(j >= 2)
            def _():
                s_wait((b + 2) % NBUF)
                if with_deg:
                    d_wait((b + 2) % NBUF)

            i_start(j + 3, (b + 3) % IDXB)
            i_wait((b + 2) % IDXB)
            g_start((b + 2) % NBUF, (b + 2) % IDXB)
            g_wait(b % NBUF)
            s_start(b % NBUF, b % IDXB)
            if with_deg:
                d_start(b % NBUF, b % IDXB)

    for j in range(N_MAIN, N_FULL):
        b = j % NBUF
        bi = j % IDXB
        s_wait((b + 2) % NBUF)
        if with_deg:
            d_wait((b + 2) % NBUF)
        if j < N_FULL - 3:
            i_start(j + 3, (bi + 3) % IDXB)
        if j < N_FULL - 2:
            i_wait((bi + 2) % IDXB)
            g_start((b + 2) % NBUF, (bi + 2) % IDXB)
        g_wait(b)
        s_start(b, bi)
        if with_deg:
            d_start(b, bi)

    t_is.wait()
    t_id.wait()
    t_g.start()

    for j in (N_FULL - 2, N_FULL - 1):
        s_wait(j % NBUF)
        if with_deg:
            d_wait(j % NBUF)

    t_g.wait()
    pltpu.sync_copy(rowr, acc_s.at[idxd_r], add=True)
    if with_deg:
        pltpu.sync_copy(onesb.at[pl.ds(0, TAIL)], deg_s.at[idxd_r], add=True)

    plsc.subcore_barrier()

    @pl.when(c == 0)
    def _():
        pltpu.sync_copy(acc_s.at[rslc], acc0_hbm.at[rslc])
        if with_deg:
            pltpu.sync_copy(deg_s.at[rslc], deg0_hbm.at[rslc])

    @pl.when(c == 1)
    def _():
        pltpu.sync_copy(acc_s.at[rslc], acc1_hbm.at[rslc])
        if with_deg:
            pltpu.sync_copy(deg_s.at[rslc], deg1_hbm.at[rslc])


def _make_sc_layer(with_deg):
    mesh = plsc.VectorSubcoreMesh(core_axis_name="c", subcore_axis_name="s")
    f32 = jnp.float32
    i32 = jnp.int32
    out_type = [jax.ShapeDtypeStruct((N_PAD, DIM), f32),
                jax.ShapeDtypeStruct((N_PAD, DIM), f32)]
    scratch = [pltpu.VMEM_SHARED((N_PAD, DIM), f32)]
    if with_deg:
        out_type += [jax.ShapeDtypeStruct((N_PAD, DEG_W), f32),
                     jax.ShapeDtypeStruct((N_PAD, DEG_W), f32)]
        scratch += [pltpu.VMEM_SHARED((N_PAD, DEG_W), f32)]
    scratch += [pltpu.VMEM((IDXB, CHUNK), i32),
                pltpu.VMEM((IDXB, CHUNK), i32),
                pltpu.VMEM((NBUF, CHUNK, DIM), f32)]
    if with_deg:
        scratch += [pltpu.VMEM((CHUNK, DEG_W), f32)]
    scratch += [pltpu.VMEM((TAIL,), i32),
                pltpu.VMEM((TAIL,), i32),
                pltpu.VMEM((TAIL, DIM), f32)]
    nsem = (3 * NBUF if with_deg else 2 * NBUF) + IDXB + 1
    scratch += [pltpu.SemaphoreType.DMA] * nsem
    return pl.kernel(functools.partial(_sc_layer_body, with_deg),
                     out_type=out_type, mesh=mesh, scratch_types=scratch,
                     compiler_params=pltpu.CompilerParams(
                         use_tc_tiling_on_sc=False),
                     name="sc_gnn_layer_deg" if with_deg else "sc_gnn_layer")


_sc_layer_with_deg = _make_sc_layer(True)
_sc_layer = _make_sc_layer(False)


def _tc_body(relu, acc0, acc1, deg0, deg1, w0, b0, w1, b1, out):
    inv0 = 1.0 / jnp.maximum(deg0[...][:, 0:1], 1.0)
    inv1 = 1.0 / jnp.maximum(deg1[...][:, 0:1], 1.0)
    y = (jnp.dot(acc0[...] * inv0, w0[...], preferred_element_type=jnp.float32)
         + jnp.dot(acc1[...] * inv1, w1[...], preferred_element_type=jnp.float32)
         + b0[...] + b1[...])
    if relu:
        y = jnp.maximum(y, 0.0)
    out[...] = y


_TC_ROWS = 1000


def _tc_layer(relu, acc0, acc1, deg0, deg1, w0, b0, w1, b1):
    grid = (N_NODES // _TC_ROWS,)
    rb = lambda i: (i, 0)
    fix = lambda i: (0, 0)
    return pl.pallas_call(
        functools.partial(_tc_body, relu),
        grid=grid,
        in_specs=[
            pl.BlockSpec((_TC_ROWS, DIM), rb),
            pl.BlockSpec((_TC_ROWS, DIM), rb),
            pl.BlockSpec((_TC_ROWS, DEG_W), rb),
            pl.BlockSpec((_TC_ROWS, DEG_W), rb),
            pl.BlockSpec((DIM, DIM), fix),
            pl.BlockSpec((1, DIM), fix),
            pl.BlockSpec((DIM, DIM), fix),
            pl.BlockSpec((1, DIM), fix),
        ],
        out_specs=pl.BlockSpec((_TC_ROWS, DIM), rb),
        out_shape=jax.ShapeDtypeStruct((N_NODES, DIM), jnp.float32),
    )(acc0, acc1, deg0, deg1, w0, b0, w1, b1)


def kernel(x, edge_index_rel0, edge_index_rel1, W0_0, b0_0, W0_1, b0_1,
           W1_0, b1_0, W1_1, b1_1, W2_0, b2_0, W2_1, b2_1):
    f32 = jnp.float32
    src = jnp.concatenate([edge_index_rel0[0], edge_index_rel1[0]])
    dst = jnp.concatenate([edge_index_rel0[1], edge_index_rel1[1]])
    zacc = jnp.zeros((N_PAD, DIM), f32)
    zdeg = jnp.zeros((N_PAD, DEG_W), f32)
    ones = jnp.ones((CHUNK, DEG_W), f32)

    acc0, acc1, deg0, deg1 = _sc_layer_with_deg(x, src, dst, zacc, zdeg, ones)
    h = _tc_layer(True, acc0, acc1, deg0, deg1,
                  W0_0, b0_0.reshape(1, DIM), W0_1, b0_1.reshape(1, DIM))

    acc0, acc1 = _sc_layer(h, src, dst, zacc)
    h = _tc_layer(True, acc0, acc1, deg0, deg1,
                  W1_0, b1_0.reshape(1, DIM), W1_1, b1_1.reshape(1, DIM))

    acc0, acc1 = _sc_layer(h, src, dst, zacc)
    h = _tc_layer(False, acc0, acc1, deg0, deg1,
                  W2_0, b2_0.reshape(1, DIM), W2_1, b2_1.reshape(1, DIM))
    return h


# speedup vs baseline: 11.2389x; 1.0044x over previous
import functools

import jax
import jax.numpy as jnp
from jax import lax
from jax.experimental import pallas as pl
from jax.experimental.pallas import tpu as pltpu
from jax.experimental.pallas import tpu_sc as plsc

N_NODES = 10000
N_PAD = 10240
E_PER_REL = 160000
DIM = 128

NC = 2
NS = 16
EDGES_PER_TILE = E_PER_REL // NS
ROWS_PER_TILE = N_PAD // NS
DEG_W = 16

CHUNK_DEG = 64
CHUNK_ND = 88
NBUF = 4
IDXB = 8
UNROLL = 8


def _sc_layer_body(with_deg, CHUNK, *refs):
    N_FULL = EDGES_PER_TILE // CHUNK
    TAIL = EDGES_PER_TILE - N_FULL * CHUNK
    N_MAIN = ((N_FULL - 3) // UNROLL) * UNROLL
    if with_deg:
        (h_hbm, src_hbm, dst_hbm, zacc_hbm, zdeg_hbm, ones_hbm,
         acc0_hbm, acc1_hbm, deg0_hbm, deg1_hbm,
         acc_s, deg_s, idxs_v, idxd_v, rowb, onesb,
         idxs_r, idxd_r, *sems) = refs
    else:
        (h_hbm, src_hbm, dst_hbm, zacc_hbm,
         acc0_hbm, acc1_hbm,
         acc_s, idxs_v, idxd_v, rowb,
         idxs_r, idxd_r, *sems) = refs
        deg_s = zdeg_hbm = ones_hbm = onesb = None
    gsem = sems[:NBUF]
    ssem = sems[NBUF:2 * NBUF]
    isem = sems[2 * NBUF:2 * NBUF + IDXB]
    zsem = sems[2 * NBUF + IDXB]
    dsem = sems[2 * NBUF + IDXB + 1:]

    c = lax.axis_index("c")
    s = lax.axis_index("s")
    ebase = (c * NS + s) * EDGES_PER_TILE

    rslc = pl.ds(s * ROWS_PER_TILE, ROWS_PER_TILE)
    zdescs = [pltpu.make_async_copy(zacc_hbm.at[rslc], acc_s.at[rslc], zsem)]
    if with_deg:
        zdescs.append(pltpu.make_async_copy(zdeg_hbm.at[rslc],
                                            deg_s.at[rslc], zsem))
        zdescs.append(pltpu.make_async_copy(ones_hbm, onesb, zsem))
    for zd in zdescs:
        zd.start()

    def i_start(k, bi):
        off = ebase + k * CHUNK
        pltpu.make_async_copy(src_hbm.at[pl.ds(off, CHUNK)], idxs_v.at[bi],
                              isem[bi]).start()
        pltpu.make_async_copy(dst_hbm.at[pl.ds(off, CHUNK)], idxd_v.at[bi],
                              isem[bi]).start()

    def i_wait(bi):
        pltpu.make_async_copy(src_hbm.at[pl.ds(0, CHUNK)], idxs_v.at[bi],
                              isem[bi]).wait()
        pltpu.make_async_copy(dst_hbm.at[pl.ds(0, CHUNK)], idxd_v.at[bi],
                              isem[bi]).wait()

    def g_start(b, bi):
        pltpu.make_async_copy(h_hbm.at[idxs_v.at[bi]], rowb.at[b],
                              gsem[b]).start()

    def g_wait(b):
        pltpu.make_async_copy(h_hbm.at[idxs_v.at[0]], rowb.at[b],
                              gsem[b]).wait()

    def s_start(b, bi):
        pltpu.make_async_copy(rowb.at[b], acc_s.at[idxd_v.at[bi]],
                              ssem[b]).start(add=True)

    def s_wait(b):
        pltpu.make_async_copy(rowb.at[b], acc_s.at[idxd_v.at[0]],
                              ssem[b]).wait()

    def d_start(b, bi):
        pltpu.make_async_copy(onesb, deg_s.at[idxd_v.at[bi]],
                              dsem[b]).start(add=True)

    def d_wait(b):
        pltpu.make_async_copy(onesb, deg_s.at[idxd_v.at[0]],
                              dsem[b]).wait()

    toff = ebase + N_FULL * CHUNK
    t_is = pltpu.make_async_copy(src_hbm.at[pl.ds(toff, TAIL)], idxs_r, zsem)
    t_id = pltpu.make_async_copy(dst_hbm.at[pl.ds(toff, TAIL)], idxd_r, zsem)
    trow = rowb.at[(N_FULL - 3) % NBUF, pl.ds(0, TAIL)]
    t_g = pltpu.make_async_copy(h_hbm.at[idxs_r], trow, zsem)
    t_is.start()
    t_id.start()
    for k in range(3):
        i_start(k, k)
    i_wait(0)
    g_start(0, 0)
    i_wait(1)
    g_start(1, 1)
    for zd in zdescs:
        zd.wait()
    plsc.subcore_barrier()

    @pl.loop(0, N_MAIN // UNROLL)
    def _outer(g):
        j0 = g * UNROLL
        for b in range(UNROLL):
            j = j0 + b
            @pl.when(j >= 2)
            def _():
                s_wait((b + 2) % NBUF)
                if with_deg:
                    d_wait((b + 2) % NBUF)

            i_start(j + 3, (b + 3) % IDXB)
            i_wait((b + 2) % IDXB)
            g_start((b + 2) % NBUF, (b + 2) % IDXB)
            g_wait(b % NBUF)
            s_start(b % NBUF, b % IDXB)
            if with_deg:
                d_start(b % NBUF, b % IDXB)

    for j in range(N_MAIN, N_FULL):
        b = j % NBUF
        bi = j % IDXB
        s_wait((b + 2) % NBUF)
        if with_deg:
            d_wait((b + 2) % NBUF)
        if j < N_FULL - 3:
            i_start(j + 3, (bi + 3) % IDXB)
        if j < N_FULL - 2:
            i_wait((bi + 2) % IDXB)
            g_start((b + 2) % NBUF, (bi + 2) % IDXB)
        g_wait(b)
        s_start(b, bi)
        if with_deg:
            d_start(b, bi)

    t_is.wait()
    t_id.wait()
    t_g.start()

    for j in (N_FULL - 2, N_FULL - 1):
        s_wait(j % NBUF)
        if with_deg:
            d_wait(j % NBUF)

    t_g.wait()
    pltpu.sync_copy(trow, acc_s.at[idxd_r], add=True)
    if with_deg:
        pltpu.sync_copy(onesb.at[pl.ds(0, TAIL)], deg_s.at[idxd_r], add=True)

    plsc.subcore_barrier()

    @pl.when(c == 0)
    def _():
        pltpu.sync_copy(acc_s.at[rslc], acc0_hbm.at[rslc])
        if with_deg:
            pltpu.sync_copy(deg_s.at[rslc], deg0_hbm.at[rslc])

    @pl.when(c == 1)
    def _():
        pltpu.sync_copy(acc_s.at[rslc], acc1_hbm.at[rslc])
        if with_deg:
            pltpu.sync_copy(deg_s.at[rslc], deg1_hbm.at[rslc])


def _make_sc_layer(with_deg):
    CHUNK = CHUNK_DEG if with_deg else CHUNK_ND
    TAIL = EDGES_PER_TILE - (EDGES_PER_TILE // CHUNK) * CHUNK
    mesh = plsc.VectorSubcoreMesh(core_axis_name="c", subcore_axis_name="s")
    f32 = jnp.float32
    i32 = jnp.int32
    out_type = [jax.ShapeDtypeStruct((N_PAD, DIM), f32),
                jax.ShapeDtypeStruct((N_PAD, DIM), f32)]
    scratch = [pltpu.VMEM_SHARED((N_PAD, DIM), f32)]
    if with_deg:
        out_type += [jax.ShapeDtypeStruct((N_PAD, DEG_W), f32),
                     jax.ShapeDtypeStruct((N_PAD, DEG_W), f32)]
        scratch += [pltpu.VMEM_SHARED((N_PAD, DEG_W), f32)]
    scratch += [pltpu.VMEM((IDXB, CHUNK), i32),
                pltpu.VMEM((IDXB, CHUNK), i32),
                pltpu.VMEM((NBUF, CHUNK, DIM), f32)]
    if with_deg:
        scratch += [pltpu.VMEM((CHUNK, DEG_W), f32)]
    scratch += [pltpu.VMEM((TAIL,), i32),
                pltpu.VMEM((TAIL,), i32)]
    nsem = (3 * NBUF if with_deg else 2 * NBUF) + IDXB + 1
    scratch += [pltpu.SemaphoreType.DMA] * nsem
    return pl.kernel(functools.partial(_sc_layer_body, with_deg, CHUNK),
                     out_type=out_type, mesh=mesh, scratch_types=scratch,
                     compiler_params=pltpu.CompilerParams(
                         use_tc_tiling_on_sc=False),
                     name="sc_gnn_layer_deg" if with_deg else "sc_gnn_layer")


_sc_layer_with_deg = _make_sc_layer(True)
_sc_layer = _make_sc_layer(False)


def _tc_body(relu, acc0, acc1, deg0, deg1, w0, b0, w1, b1, out):
    inv0 = 1.0 / jnp.maximum(deg0[...][:, 0:1], 1.0)
    inv1 = 1.0 / jnp.maximum(deg1[...][:, 0:1], 1.0)
    y = (jnp.dot(acc0[...] * inv0, w0[...], preferred_element_type=jnp.float32)
         + jnp.dot(acc1[...] * inv1, w1[...], preferred_element_type=jnp.float32)
         + b0[...] + b1[...])
    if relu:
        y = jnp.maximum(y, 0.0)
    out[...] = y


_TC_ROWS = 1000


def _tc_layer(relu, acc0, acc1, deg0, deg1, w0, b0, w1, b1):
    grid = (N_NODES // _TC_ROWS,)
    rb = lambda i: (i, 0)
    fix = lambda i: (0, 0)
    return pl.pallas_call(
        functools.partial(_tc_body, relu),
        grid=grid,
        in_specs=[
            pl.BlockSpec((_TC_ROWS, DIM), rb),
            pl.BlockSpec((_TC_ROWS, DIM), rb),
            pl.BlockSpec((_TC_ROWS, DEG_W), rb),
            pl.BlockSpec((_TC_ROWS, DEG_W), rb),
            pl.BlockSpec((DIM, DIM), fix),
            pl.BlockSpec((1, DIM), fix),
            pl.BlockSpec((DIM, DIM), fix),
            pl.BlockSpec((1, DIM), fix),
        ],
        out_specs=pl.BlockSpec((_TC_ROWS, DIM), rb),
        out_shape=jax.ShapeDtypeStruct((N_NODES, DIM), jnp.float32),
    )(acc0, acc1, deg0, deg1, w0, b0, w1, b1)


def kernel(x, edge_index_rel0, edge_index_rel1, W0_0, b0_0, W0_1, b0_1,
           W1_0, b1_0, W1_1, b1_1, W2_0, b2_0, W2_1, b2_1):
    f32 = jnp.float32
    src = jnp.concatenate([edge_index_rel0[0], edge_index_rel1[0]])
    dst = jnp.concatenate([edge_index_rel0[1], edge_index_rel1[1]])
    zacc = jnp.zeros((N_PAD, DIM), f32)
    zdeg = jnp.zeros((N_PAD, DEG_W), f32)
    ones = jnp.ones((CHUNK_DEG, DEG_W), f32)

    acc0, acc1, deg0, deg1 = _sc_layer_with_deg(x, src, dst, zacc, zdeg, ones)
    h = _tc_layer(True, acc0, acc1, deg0, deg1,
                  W0_0, b0_0.reshape(1, DIM), W0_1, b0_1.reshape(1, DIM))

    acc0, acc1 = _sc_layer(h, src, dst, zacc)
    h = _tc_layer(True, acc0, acc1, deg0, deg1,
                  W1_0, b1_0.reshape(1, DIM), W1_1, b1_1.reshape(1, DIM))

    acc0, acc1 = _sc_layer(h, src, dst, zacc)
    h = _tc_layer(False, acc0, acc1, deg0, deg1,
                  W2_0, b2_0.reshape(1, DIM), W2_1, b2_1.reshape(1, DIM))
    return h
